# baseline (speedup 1.0000x reference)
import jax
import jax.numpy as jnp
from jax.experimental import pallas as pl
from jax.experimental.pallas import tpu as pltpu

_STRIDE = 8.0
_H = 128
_W = 128
_NA = 3
_NCLS = 80
_YC = 32
_CPP = _H // _YC
_L = _YC * _W


def _decode_body(bbox_ref, conf_ref, cls_ref, anchors_ref,
                 xywha_ref, idx_ref, confs_ref):
    a = pl.program_id(0)
    q = pl.program_id(1)

    t = bbox_ref[:, 0]
    e = jnp.exp(t)
    sig = e * (1.0 / (1.0 + e))
    comp = jax.lax.broadcasted_iota(jnp.int32, (1, 5, 1, 1), 1)
    xf = jax.lax.broadcasted_iota(jnp.int32, (1, 1, 1, _W), 3).astype(jnp.float32)
    yf = (jax.lax.broadcasted_iota(jnp.int32, (1, 1, _YC, 1), 2)
          + q * _YC).astype(jnp.float32)
    mesh = jnp.where(comp == 0, xf, jnp.where(comp == 1, yf, 0.0))
    aw = jnp.where(a == 0, anchors_ref[0, 0],
                   jnp.where(a == 1, anchors_ref[1, 0], anchors_ref[2, 0]))
    ah = jnp.where(a == 0, anchors_ref[0, 1],
                   jnp.where(a == 1, anchors_ref[1, 1], anchors_ref[2, 1]))
    anch = jnp.where(comp == 2, aw, ah)
    xy = (sig + mesh) * _STRIDE
    ang = sig * 360.0 - 180.0
    out = jnp.where(comp < 2, xy, jnp.where(comp == 4, ang, e * anch))
    xywha_ref[...] = out.transpose(1, 0, 2, 3).reshape(5, 8, _L)

    c = cls_ref[:, 0]
    m = jnp.max(c, axis=2, keepdims=True)
    sub = jax.lax.broadcasted_iota(jnp.int32, (1, 1, _NCLS, 1), 2)
    first_max = jnp.min(jnp.where(c == m, sub, jnp.int32(_NCLS)),
                        axis=2)
    idx_ref[...] = first_max.reshape(8, _L)

    cf = conf_ref[:, 0]
    em = jnp.exp(m[:, :, 0, :])
    ec = jnp.exp(cf)
    confs_ref[...] = ((em * ec) * (1.0 / ((1.0 + em) * (1.0 + ec)))).reshape(8, _L)


def kernel(bbox, conf, cls, anchors, img_size):
    nB, nA, nH, nW, _ = bbox.shape
    n_cls = cls.shape[-1]
    flat = nA * nH * nW

    bbox_t = jnp.transpose(bbox, (0, 1, 4, 2, 3))
    conf_s = conf.reshape(nB, nA, nH, nW)
    cls_t = jnp.transpose(cls, (0, 1, 2, 4, 3))

    xywha, idx, confs = pl.pallas_call(
        _decode_body,
        grid=(nA, _CPP),
        in_specs=[
            pl.BlockSpec((nB, 1, 5, _YC, nW), lambda a, q: (0, a, 0, q, 0)),
            pl.BlockSpec((nB, 1, _YC, nW), lambda a, q: (0, a, q, 0)),
            pl.BlockSpec((nB, 1, _YC, n_cls, nW), lambda a, q: (0, a, q, 0, 0)),
            pl.BlockSpec((_NA, 2), lambda a, q: (0, 0)),
        ],
        out_specs=[
            pl.BlockSpec((5, nB, _L), lambda a, q: (0, 0, a * _CPP + q)),
            pl.BlockSpec((nB, _L), lambda a, q: (0, a * _CPP + q)),
            pl.BlockSpec((nB, _L), lambda a, q: (0, a * _CPP + q)),
        ],
        out_shape=[
            jax.ShapeDtypeStruct((5, nB, flat), jnp.float32),
            jax.ShapeDtypeStruct((nB, flat), jnp.int32),
            jax.ShapeDtypeStruct((nB, flat), jnp.float32),
        ],
        compiler_params=pltpu.CompilerParams(
            dimension_semantics=("arbitrary", "arbitrary"),
        ),
    )(bbox_t, conf_s, cls_t, anchors)

    return (jnp.transpose(xywha, (1, 2, 0)), idx, confs)


# speedup vs baseline: 1.0023x; 1.0023x over previous
import jax
import jax.numpy as jnp
from jax.experimental import pallas as pl
from jax.experimental.pallas import tpu as pltpu

_STRIDE = 8.0
_H = 128
_W = 128
_NA = 3
_NCLS = 80
_YC = 32
_CPP = _H // _YC
_L = _YC * _W


def _decode_body(bbox_ref, conf_ref, cls_ref, anchors_ref,
                 xywha_ref, idx_ref, confs_ref):
    a = pl.program_id(0)
    q = pl.program_id(1)

    t = bbox_ref[:, 0]
    e = jnp.exp(t)
    sig = e * (1.0 / (1.0 + e))
    comp = jax.lax.broadcasted_iota(jnp.int32, (1, 5, 1, 1), 1)
    xf = jax.lax.broadcasted_iota(jnp.int32, (1, 1, 1, _W), 3).astype(jnp.float32)
    yf = (jax.lax.broadcasted_iota(jnp.int32, (1, 1, _YC, 1), 2)
          + q * _YC).astype(jnp.float32)
    mesh = jnp.where(comp == 0, xf, jnp.where(comp == 1, yf, 0.0))
    aw = jnp.where(a == 0, anchors_ref[0, 0],
                   jnp.where(a == 1, anchors_ref[1, 0], anchors_ref[2, 0]))
    ah = jnp.where(a == 0, anchors_ref[0, 1],
                   jnp.where(a == 1, anchors_ref[1, 1], anchors_ref[2, 1]))
    anch = jnp.where(comp == 2, aw, ah)
    xy = (sig + mesh) * _STRIDE
    ang = sig * 360.0 - 180.0
    out = jnp.where(comp < 2, xy, jnp.where(comp == 4, ang, e * anch))
    xywha_ref[...] = out.transpose(1, 0, 2, 3).reshape(5, 8, _L)

    c = cls_ref[:, 0]
    m = jnp.max(c, axis=2, keepdims=True)
    sub = jax.lax.broadcasted_iota(jnp.int32, (1, 1, _NCLS, 1), 2)
    first_max = jnp.min(jnp.where(c == m, sub, jnp.int32(_NCLS)),
                        axis=2)
    idx_ref[...] = first_max.reshape(8, _L)

    cf = conf_ref[:, 0]
    em = jnp.exp(m[:, :, 0, :])
    ec = jnp.exp(cf)
    confs_ref[...] = ((em * ec) * (1.0 / ((1.0 + em) * (1.0 + ec)))).reshape(8, _L)


def kernel(bbox, conf, cls, anchors, img_size):
    nB, nA, nH, nW, _ = bbox.shape
    n_cls = cls.shape[-1]
    flat = nA * nH * nW

    bbox_t = jnp.transpose(bbox, (0, 1, 4, 2, 3))
    conf_s = conf.reshape(nB, nA, nH, nW)
    cls_t = jnp.transpose(cls, (0, 1, 2, 4, 3))

    xywha, idx, confs = pl.pallas_call(
        _decode_body,
        grid=(nA, _CPP),
        in_specs=[
            pl.BlockSpec((nB, 1, 5, _YC, nW), lambda a, q: (0, a, 0, q, 0)),
            pl.BlockSpec((nB, 1, _YC, nW), lambda a, q: (0, a, q, 0)),
            pl.BlockSpec((nB, 1, _YC, n_cls, nW), lambda a, q: (0, a, q, 0, 0)),
            pl.BlockSpec((_NA, 2), lambda a, q: (0, 0)),
        ],
        out_specs=[
            pl.BlockSpec((5, nB, _L), lambda a, q: (0, 0, a * _CPP + q)),
            pl.BlockSpec((nB, _L), lambda a, q: (0, a * _CPP + q)),
            pl.BlockSpec((nB, _L), lambda a, q: (0, a * _CPP + q)),
        ],
        out_shape=[
            jax.ShapeDtypeStruct((5, nB, flat), jnp.float32),
            jax.ShapeDtypeStruct((nB, flat), jnp.int32),
            jax.ShapeDtypeStruct((nB, flat), jnp.float32),
        ],
        compiler_params=pltpu.CompilerParams(
            dimension_semantics=("parallel", "parallel"),
        ),
    )(bbox_t, conf_s, cls_t, anchors)

    return (jnp.transpose(xywha, (1, 2, 0)), idx, confs)
